# baseline (speedup 1.0000x reference)
import functools

import jax
import jax.numpy as jnp
from jax import lax
from jax.experimental import pallas as pl
from jax.experimental.pallas import tpu as pltpu
from jax.experimental.pallas import tpu_sc as plsc

N_NODES = 10000
N_EDGES = 320000
D_FEAT = 128

NC = 2
NS = 16
NW = NC * NS
EPW = N_EDGES // NW
NPAD = 10240
ZSL = NPAD // NS

BLK = 16000
COLS = BLK // 8
CHUNK0 = 192000
CHUNK1 = N_EDGES - CHUNK0

_EPS = 1e-12


def _alpha_body(h_ref, t_ref, o_ref):
    h = h_ref[...]
    t = t_ref[...]
    ones = jnp.ones((D_FEAT, 1), jnp.float32)
    dot = jax.lax.dot(h * t, ones).reshape(o_ref.shape)
    hs = jax.lax.dot(h * h, ones).reshape(o_ref.shape)
    ts = jax.lax.dot(t * t, ones).reshape(o_ref.shape)
    denom_sq = jnp.maximum(hs, _EPS * _EPS) * jnp.maximum(ts, _EPS * _EPS)
    o_ref[...] = dot * (0.5 * lax.rsqrt(denom_sq)) + 0.5


def _alpha_tc(head_embeds, tail_embeds, n_edges, block_off):
    grid = (n_edges // BLK,)
    return pl.pallas_call(
        _alpha_body,
        grid=grid,
        in_specs=[
            pl.BlockSpec((BLK, D_FEAT), lambda i: (i + block_off, 0)),
            pl.BlockSpec((BLK, D_FEAT), lambda i: (i + block_off, 0)),
        ],
        out_specs=pl.BlockSpec((8, COLS), lambda i: (i, 0)),
        out_shape=jax.ShapeDtypeStruct((n_edges // COLS, COLS), jnp.float32),
    )(head_embeds, tail_embeds)


def _make_segment_sum_body(epw_c, idx_base):
    def body(idx_hbm, alpha_hbm, out_hbm, idx_v, alpha_v, zbuf, d_sh,
             sem_i, sem_a):
        c = lax.axis_index("c")
        s = lax.axis_index("s")
        wid = s * NC + c

        cp_i = pltpu.async_copy(
            idx_hbm.at[pl.ds(idx_base + wid * epw_c, epw_c)], idx_v, sem_i)
        cp_a = pltpu.async_copy(
            alpha_hbm.at[pl.ds(wid * epw_c, epw_c)], alpha_v, sem_a)

        @plsc.parallel_loop(0, ZSL, step=16, unroll=4)
        def zero_body(i):
            zbuf[pl.ds(i, 16)] = jnp.zeros((16,), jnp.float32)

        pltpu.sync_copy(zbuf, d_sh.at[pl.ds(s * ZSL, ZSL)])
        cp_i.wait()
        cp_a.wait()
        plsc.subcore_barrier()
        pltpu.sync_copy(alpha_v, d_sh.at[idx_v], add=True)
        plsc.subcore_barrier()

        @pl.when(s == 0)
        def _():
            pltpu.sync_copy(d_sh, out_hbm.at[pl.ds(c * NPAD, NPAD)])

    return body


def _finalize_body(dp0_hbm, dp1_hbm, idx_hbm, alpha_hbm, out_hbm, a_v, b_v,
                   d_full, idx_v, alpha_v, out_v, d_sh, sem_i, sem_a):
    c = lax.axis_index("c")
    s = lax.axis_index("s")
    wid = s * NC + c
    base = wid * EPW

    cp_i = pltpu.async_copy(idx_hbm.at[pl.ds(base, EPW)], idx_v, sem_i)
    cp_a = pltpu.async_copy(alpha_hbm.at[pl.ds(base, EPW)], alpha_v, sem_a)

    sl_lo = pl.ds(s * ZSL, ZSL)
    sl_hi = pl.ds(NPAD + s * ZSL, ZSL)
    pltpu.sync_copy(dp0_hbm.at[sl_lo], a_v)
    pltpu.sync_copy(dp0_hbm.at[sl_hi], b_v)

    @plsc.parallel_loop(0, ZSL, step=16, unroll=4)
    def sum01_body(i):
        sl = pl.ds(i, 16)
        a_v[sl] = a_v[sl] + b_v[sl]

    pltpu.sync_copy(dp1_hbm.at[sl_lo], b_v)

    @plsc.parallel_loop(0, ZSL, step=16, unroll=4)
    def sum2_body(i):
        sl = pl.ds(i, 16)
        a_v[sl] = a_v[sl] + b_v[sl]

    pltpu.sync_copy(dp1_hbm.at[sl_hi], b_v)

    @plsc.parallel_loop(0, ZSL, step=16, unroll=4)
    def inv_body(i):
        sl = pl.ds(i, 16)
        dsum = a_v[sl] + b_v[sl]
        a_v[sl] = jnp.where(dsum != 0.0, 1.0 / dsum, 0.0)

    pltpu.sync_copy(a_v, d_sh.at[pl.ds(s * ZSL, ZSL)])
    plsc.subcore_barrier()
    pltpu.sync_copy(d_sh, d_full)
    cp_i.wait()
    cp_a.wait()

    @plsc.parallel_loop(0, EPW, step=16, unroll=4)
    def gather_body(i):
        sl = pl.ds(i, 16)
        out_v[sl] = plsc.load_gather(d_full, [idx_v[sl]]) * alpha_v[sl]

    pltpu.sync_copy(out_v, out_hbm.at[pl.ds(base, EPW)])


@functools.lru_cache(maxsize=1)
def _sc_kernels():
    mesh = plsc.VectorSubcoreMesh(core_axis_name="c", subcore_axis_name="s")
    params = pltpu.CompilerParams(needs_layout_passes=False)

    def make_segsum(epw_c, idx_base):
        return pl.kernel(
            _make_segment_sum_body(epw_c, idx_base),
            out_type=jax.ShapeDtypeStruct((NC * NPAD,), jnp.float32),
            mesh=mesh,
            compiler_params=params,
            scratch_types=[
                pltpu.VMEM((epw_c,), jnp.int32),
                pltpu.VMEM((epw_c,), jnp.float32),
                pltpu.VMEM((ZSL,), jnp.float32),
                pltpu.VMEM_SHARED((NPAD,), jnp.float32),
                pltpu.SemaphoreType.DMA,
                pltpu.SemaphoreType.DMA,
            ],
        )

    segsum0 = make_segsum(CHUNK0 // NW, 0)
    segsum1 = make_segsum(CHUNK1 // NW, CHUNK0)
    finalize = pl.kernel(
        _finalize_body,
        out_type=jax.ShapeDtypeStruct((N_EDGES,), jnp.float32),
        mesh=mesh,
        compiler_params=params,
        scratch_types=[
            pltpu.VMEM((ZSL,), jnp.float32),
            pltpu.VMEM((ZSL,), jnp.float32),
            pltpu.VMEM((NPAD,), jnp.float32),
            pltpu.VMEM((EPW,), jnp.int32),
            pltpu.VMEM((EPW,), jnp.float32),
            pltpu.VMEM((EPW,), jnp.float32),
            pltpu.VMEM_SHARED((NPAD,), jnp.float32),
            pltpu.SemaphoreType.DMA,
            pltpu.SemaphoreType.DMA,
        ],
    )
    return segsum0, segsum1, finalize


def kernel(head_embeds, tail_embeds, head_list, tail_list):
    segsum0, segsum1, finalize_sc = _sc_kernels()
    alpha0 = _alpha_tc(head_embeds, tail_embeds, CHUNK0, 0).reshape(CHUNK0)
    dp0 = segsum0(head_list, alpha0)
    alpha1 = _alpha_tc(head_embeds, tail_embeds, CHUNK1,
                       CHUNK0 // BLK).reshape(CHUNK1)
    dp1 = segsum1(head_list, alpha1)
    alpha = jnp.concatenate([alpha0, alpha1])
    g_values = finalize_sc(dp0, dp1, head_list, alpha)
    g_indices = jnp.stack([head_list, tail_list], axis=0)
    return (g_indices, g_values)
